# Initial kernel scaffold; baseline (speedup 1.0000x reference)
import jax
import jax.numpy as jnp
from jax.experimental import pallas as pl


def kernel(x_base, x_centroid, edge_attr_bb, edge_attr_bc, edge_attr_cc, edge_attr_cb, edge_weight_bb, edge_weight_bc, edge_weight_cc, edge_weight_cb, src_bb, dst_bb, src_bc, dst_bc, src_cc, dst_cc, src_cb, dst_cb, W_src, b_src, W_dst, b_dst, W_edge, b_edge, eps, W_mlp, b_mlp, ln_g, ln_b):
    raise NotImplementedError("write your pallas kernel here")


# speedup vs baseline: 1.2560x; 1.2560x over previous
import functools

import jax
import jax.numpy as jnp
from jax import lax
from jax.experimental import pallas as pl
from jax.experimental.pallas import tpu as pltpu
from jax.experimental.pallas import tpu_sc as plsc

HID = 256
NB = 10000
NCN = 1000
DE = 16
L = 2
CONVS = [("bb", "base", "base"), ("bc", "base", "centroid"),
         ("cc", "centroid", "centroid"), ("cb", "centroid", "base")]
EDGE_N = {"bb": 160000, "bc": 40000, "cc": 32000, "cb": 40000}

SC_CORES = 2
SC_SUBCORES = 16
CHUNK = 128
PAD_UNIT = SC_CORES * SC_SUBCORES * CHUNK

NPAD = {"base": 10240, "centroid": 1024}
NNODE = {"base": NB, "centroid": NCN}


def _ceil_to(x, m):
    return (x + m - 1) // m * m


def _lin_body(x_ref, w_ref, b_ref, o_ref):
    o_ref[...] = (jnp.dot(x_ref[...], w_ref[...],
                          preferred_element_type=jnp.float32) + b_ref[...])


def _lin(x, w, b, block_rows):
    n, k = x.shape
    m = w.shape[1]
    return pl.pallas_call(
        _lin_body,
        grid=(n // block_rows,),
        in_specs=[pl.BlockSpec((block_rows, k), lambda i: (i, 0)),
                  pl.BlockSpec((k, m), lambda i: (0, 0)),
                  pl.BlockSpec((1, m), lambda i: (0, 0))],
        out_specs=pl.BlockSpec((block_rows, m), lambda i: (i, 0)),
        out_shape=jax.ShapeDtypeStruct((n, m), jnp.float32),
    )(x, w, b.reshape(1, m))


def _msg_body(g_ref, ea_ref, ew_ref, w_ref, b_ref, o_ref):
    pre = g_ref[...] + jnp.dot(ea_ref[...], w_ref[...],
                               preferred_element_type=jnp.float32) + b_ref[...]
    o_ref[...] = jax.nn.gelu(pre) * ew_ref[...]


def _msg(g, ea, ew, w, b, block_rows):
    ep = g.shape[0]
    return pl.pallas_call(
        _msg_body,
        grid=(ep // block_rows,),
        in_specs=[pl.BlockSpec((block_rows, HID), lambda i: (i, 0)),
                  pl.BlockSpec((block_rows, DE), lambda i: (i, 0)),
                  pl.BlockSpec((block_rows, 1), lambda i: (i, 0)),
                  pl.BlockSpec((DE, HID), lambda i: (0, 0)),
                  pl.BlockSpec((1, HID), lambda i: (0, 0))],
        out_specs=pl.BlockSpec((block_rows, HID), lambda i: (i, 0)),
        out_shape=jax.ShapeDtypeStruct((ep, HID), jnp.float32),
    )(g, ea, ew, w, b.reshape(1, HID))


def _post_body(a_ref, h_ref, m0_ref, b0_ref, g_ref, be_ref, m1_ref, b1_ref,
               o_ref):
    h = a_ref[...] + h_ref[...]
    h = jnp.dot(h, m0_ref[...], preferred_element_type=jnp.float32) + b0_ref[...]
    mu = jnp.mean(h, -1, keepdims=True)
    v = jnp.mean((h - mu) ** 2, -1, keepdims=True)
    h = (h - mu) / jnp.sqrt(v + 1e-5) * g_ref[...] + be_ref[...]
    h = jax.nn.gelu(h)
    o_ref[...] = (jnp.dot(h, m1_ref[...], preferred_element_type=jnp.float32)
                  + b1_ref[...])


def _post(aggr, hd, m0, b0, lg, lb, m1, b1, block_rows):
    n = aggr.shape[0]
    full = lambda i: (0, 0)
    rows = lambda i: (i, 0)
    return pl.pallas_call(
        _post_body,
        grid=(n // block_rows,),
        in_specs=[pl.BlockSpec((block_rows, HID), rows),
                  pl.BlockSpec((block_rows, HID), rows),
                  pl.BlockSpec((HID, HID), full),
                  pl.BlockSpec((1, HID), full),
                  pl.BlockSpec((1, HID), full),
                  pl.BlockSpec((1, HID), full),
                  pl.BlockSpec((HID, HID), full),
                  pl.BlockSpec((1, HID), full)],
        out_specs=pl.BlockSpec((block_rows, HID), rows),
        out_shape=jax.ShapeDtypeStruct((n, HID), jnp.float32),
    )(aggr, hd, m0, b0.reshape(1, HID), lg.reshape(1, HID),
      lb.reshape(1, HID), m1, b1.reshape(1, HID))


def _combine_body(x_ref, a_ref, b_ref, o_ref):
    o_ref[...] = x_ref[...] + jax.nn.gelu(a_ref[...] + b_ref[...])


def _combine(x, a, b, block_rows):
    n = x.shape[0]
    rows = lambda i: (i, 0)
    return pl.pallas_call(
        _combine_body,
        grid=(n // block_rows,),
        in_specs=[pl.BlockSpec((block_rows, HID), rows)] * 3,
        out_specs=pl.BlockSpec((block_rows, HID), rows),
        out_shape=jax.ShapeDtypeStruct((n, HID), jnp.float32),
    )(x, a, b)


def _make_gather(ep):
    per_worker = ep // (SC_CORES * SC_SUBCORES)
    niter = per_worker // CHUNK
    mesh = plsc.VectorSubcoreMesh(core_axis_name="c", subcore_axis_name="s")

    @functools.partial(
        pl.kernel,
        out_type=jax.ShapeDtypeStruct((ep, HID), jnp.float32),
        mesh=mesh,
        scratch_types=[pltpu.VMEM((CHUNK,), jnp.int32),
                       pltpu.VMEM((CHUNK, HID), jnp.float32),
                       pltpu.SemaphoreType.DMA],
    )
    def gather_k(table_hbm, idx_hbm, out_hbm, idx_v, rows_v, sem):
        wid = lax.axis_index("s") * SC_CORES + lax.axis_index("c")
        base = wid * per_worker

        @pl.loop(0, niter)
        def _(j):
            off = base + j * CHUNK
            pltpu.sync_copy(idx_hbm.at[pl.ds(off, CHUNK)], idx_v)
            pltpu.async_copy(table_hbm.at[idx_v], rows_v, sem).wait()
            pltpu.sync_copy(rows_v, out_hbm.at[pl.ds(off, CHUNK)])

    return gather_k


def _make_scatter(nd_pad, ep):
    half = HID // 2
    e_sub = ep // SC_SUBCORES
    niter = e_sub // CHUNK
    rows_sub = nd_pad // SC_SUBCORES
    mesh = plsc.VectorSubcoreMesh(core_axis_name="c", subcore_axis_name="s")

    @functools.partial(
        pl.kernel,
        out_type=jax.ShapeDtypeStruct((nd_pad, HID), jnp.float32),
        mesh=mesh,
        scratch_types=[pltpu.VMEM((CHUNK,), jnp.int32),
                       pltpu.VMEM((CHUNK, half), jnp.float32),
                       pltpu.VMEM_SHARED((nd_pad, half), jnp.float32),
                       pltpu.SemaphoreType.DMA],
    )
    def scatter_k(m_hbm, di_hbm, zeros_hbm, out_hbm, idx_v, rows_v, acc_sh,
                  sem):
        cid = lax.axis_index("c")
        sid = lax.axis_index("s")
        r0 = sid * rows_sub
        col0 = cid * half
        pltpu.sync_copy(zeros_hbm.at[pl.ds(r0, rows_sub)],
                        acc_sh.at[pl.ds(r0, rows_sub)])
        plsc.subcore_barrier()

        @pl.loop(0, niter)
        def _(j):
            off = sid * e_sub + j * CHUNK
            pltpu.sync_copy(di_hbm.at[pl.ds(off, CHUNK)], idx_v)
            pltpu.sync_copy(m_hbm.at[pl.ds(off, CHUNK), pl.ds(col0, half)],
                            rows_v)
            pltpu.sync_copy(rows_v, acc_sh.at[idx_v], add=True)

        plsc.subcore_barrier()
        pltpu.sync_copy(acc_sh.at[pl.ds(r0, rows_sub)],
                        out_hbm.at[pl.ds(r0, rows_sub), pl.ds(col0, half)])

    return scatter_k


_EPAD = {k: _ceil_to(v, PAD_UNIT) for k, v in EDGE_N.items()}
_GATHER = {k: _make_gather(ep) for k, ep in _EPAD.items()}
_SCATTER = {name: _make_scatter(NPAD[dt], _EPAD[name])
            for name, _, dt in CONVS}


def kernel(x_base, x_centroid, edge_attr_bb, edge_attr_bc, edge_attr_cc,
           edge_attr_cb, edge_weight_bb, edge_weight_bc, edge_weight_cc,
           edge_weight_cb, src_bb, dst_bb, src_bc, dst_bc, src_cc, dst_cc,
           src_cb, dst_cb, W_src, b_src, W_dst, b_dst, W_edge, b_edge, eps,
           W_mlp, b_mlp, ln_g, ln_b):
    ea = {"bb": edge_attr_bb, "bc": edge_attr_bc, "cc": edge_attr_cc,
          "cb": edge_attr_cb}
    ew = {"bb": edge_weight_bb, "bc": edge_weight_bc, "cc": edge_weight_cc,
          "cb": edge_weight_cb}
    si = {"bb": src_bb, "bc": src_bc, "cc": src_cc, "cb": src_cb}
    di = {"bb": dst_bb, "bc": dst_bc, "cc": dst_cc, "cb": dst_cb}

    for name in EDGE_N:
        e = EDGE_N[name]
        ep = _EPAD[name]
        pad = ep - e
        ea[name] = jnp.pad(ea[name], ((0, pad), (0, 0)))
        ew[name] = jnp.pad(ew[name], (0, pad)).reshape(ep, 1)
        si[name] = jnp.pad(si[name], (0, pad))
        di[name] = jnp.pad(di[name], (0, pad))

    zeros = {"base": jnp.zeros((NPAD["base"], HID // 2), jnp.float32),
             "centroid": jnp.zeros((NPAD["centroid"], HID // 2), jnp.float32)}

    xmap = {"base": x_base, "centroid": x_centroid}
    for l in range(L):
        outs = {"base": [], "centroid": []}
        for c, (name, st, dt) in enumerate(CONVS):
            xs = xmap[st]
            xd = xmap[dt]
            nd = NNODE[dt]
            blk_s = 1000 if xs.shape[0] == NB else NCN
            blk_d = 1000 if nd == NB else NCN
            hsrc = _lin(xs, W_src[l, c], b_src[l, c], blk_s)
            hd = _lin(xd, W_dst[l, c] * (1.0 + eps[l, c]), b_dst[l, c], blk_d)
            g = _GATHER[name](hsrc, si[name])
            m = _msg(g, ea[name], ew[name], W_edge[l, c], b_edge[l, c], 2048)
            aggr = _SCATTER[name](m, di[name], zeros[dt])[:nd]
            out = _post(aggr, hd, W_mlp[l, c, 0], b_mlp[l, c, 0], ln_g[l, c],
                        ln_b[l, c], W_mlp[l, c, 1], b_mlp[l, c, 1], blk_d)
            outs[dt].append(out)
        xmap = {k: _combine(xmap[k], outs[k][0], outs[k][1],
                            1000 if NNODE[k] == NB else NCN)
                for k in xmap}
    return jnp.concatenate([xmap["base"], xmap["centroid"]], axis=0)


# speedup vs baseline: 1.4262x; 1.1354x over previous
import functools

import jax
import jax.numpy as jnp
from jax import lax
from jax.experimental import pallas as pl
from jax.experimental.pallas import tpu as pltpu
from jax.experimental.pallas import tpu_sc as plsc

HID = 256
NB = 10000
NCN = 1000
DE = 16
L = 2
CONVS = [("bb", "base", "base"), ("bc", "base", "centroid"),
         ("cc", "centroid", "centroid"), ("cb", "centroid", "base")]
EDGE_N = {"bb": 160000, "bc": 40000, "cc": 32000, "cb": 40000}

SC_CORES = 2
SC_SUBCORES = 16
CHUNK = 128
PAD_UNIT = SC_CORES * SC_SUBCORES * CHUNK

NPAD = {"base": 10240, "centroid": 1024}
NNODE = {"base": NB, "centroid": NCN}


def _ceil_to(x, m):
    return (x + m - 1) // m * m


def _lin_body(x_ref, w_ref, b_ref, o_ref):
    o_ref[...] = (jnp.dot(x_ref[...], w_ref[...],
                          preferred_element_type=jnp.float32) + b_ref[...])


def _lin(x, w, b, block_rows):
    n, k = x.shape
    m = w.shape[1]
    return pl.pallas_call(
        _lin_body,
        grid=(n // block_rows,),
        in_specs=[pl.BlockSpec((block_rows, k), lambda i: (i, 0)),
                  pl.BlockSpec((k, m), lambda i: (0, 0)),
                  pl.BlockSpec((1, m), lambda i: (0, 0))],
        out_specs=pl.BlockSpec((block_rows, m), lambda i: (i, 0)),
        out_shape=jax.ShapeDtypeStruct((n, m), jnp.float32),
    )(x, w, b.reshape(1, m))


def _msg_body(g_ref, ea_ref, ew_ref, w_ref, b_ref, o_ref):
    pre = g_ref[...] + jnp.dot(ea_ref[...], w_ref[...],
                               preferred_element_type=jnp.float32) + b_ref[...]
    o_ref[...] = jax.nn.gelu(pre) * ew_ref[...]


def _msg(g, ea, ew, w, b, block_rows):
    ep = g.shape[0]
    return pl.pallas_call(
        _msg_body,
        grid=(ep // block_rows,),
        in_specs=[pl.BlockSpec((block_rows, HID), lambda i: (i, 0)),
                  pl.BlockSpec((block_rows, DE), lambda i: (i, 0)),
                  pl.BlockSpec((block_rows, 1), lambda i: (i, 0)),
                  pl.BlockSpec((DE, HID), lambda i: (0, 0)),
                  pl.BlockSpec((1, HID), lambda i: (0, 0))],
        out_specs=pl.BlockSpec((block_rows, HID), lambda i: (i, 0)),
        out_shape=jax.ShapeDtypeStruct((ep, HID), jnp.float32),
    )(g, ea, ew, w, b.reshape(1, HID))


def _post_body(a_ref, h_ref, m0_ref, b0_ref, g_ref, be_ref, m1_ref, b1_ref,
               o_ref):
    h = a_ref[...] + h_ref[...]
    h = jnp.dot(h, m0_ref[...], preferred_element_type=jnp.float32) + b0_ref[...]
    mu = jnp.mean(h, -1, keepdims=True)
    v = jnp.mean((h - mu) ** 2, -1, keepdims=True)
    h = (h - mu) / jnp.sqrt(v + 1e-5) * g_ref[...] + be_ref[...]
    h = jax.nn.gelu(h)
    o_ref[...] = (jnp.dot(h, m1_ref[...], preferred_element_type=jnp.float32)
                  + b1_ref[...])


def _post(aggr, hd, m0, b0, lg, lb, m1, b1, block_rows):
    n = aggr.shape[0]
    full = lambda i: (0, 0)
    rows = lambda i: (i, 0)
    return pl.pallas_call(
        _post_body,
        grid=(n // block_rows,),
        in_specs=[pl.BlockSpec((block_rows, HID), rows),
                  pl.BlockSpec((block_rows, HID), rows),
                  pl.BlockSpec((HID, HID), full),
                  pl.BlockSpec((1, HID), full),
                  pl.BlockSpec((1, HID), full),
                  pl.BlockSpec((1, HID), full),
                  pl.BlockSpec((HID, HID), full),
                  pl.BlockSpec((1, HID), full)],
        out_specs=pl.BlockSpec((block_rows, HID), rows),
        out_shape=jax.ShapeDtypeStruct((n, HID), jnp.float32),
    )(aggr, hd, m0, b0.reshape(1, HID), lg.reshape(1, HID),
      lb.reshape(1, HID), m1, b1.reshape(1, HID))


def _combine_body(x_ref, a_ref, b_ref, o_ref):
    o_ref[...] = x_ref[...] + jax.nn.gelu(a_ref[...] + b_ref[...])


def _combine(x, a, b, block_rows):
    n = x.shape[0]
    rows = lambda i: (i, 0)
    return pl.pallas_call(
        _combine_body,
        grid=(n // block_rows,),
        in_specs=[pl.BlockSpec((block_rows, HID), rows)] * 3,
        out_specs=pl.BlockSpec((block_rows, HID), rows),
        out_shape=jax.ShapeDtypeStruct((n, HID), jnp.float32),
    )(x, a, b)


def _make_gather(ep):
    per_worker = ep // (SC_CORES * SC_SUBCORES)
    niter = per_worker // CHUNK
    mesh = plsc.VectorSubcoreMesh(core_axis_name="c", subcore_axis_name="s")

    @functools.partial(
        pl.kernel,
        out_type=jax.ShapeDtypeStruct((ep, HID), jnp.float32),
        mesh=mesh,
        scratch_types=[pltpu.VMEM((per_worker,), jnp.int32),
                       pltpu.VMEM((CHUNK, HID), jnp.float32),
                       pltpu.VMEM((CHUNK, HID), jnp.float32),
                       pltpu.SemaphoreType.DMA,
                       pltpu.SemaphoreType.DMA,
                       pltpu.SemaphoreType.DMA,
                       pltpu.SemaphoreType.DMA],
    )
    def gather_k(table_hbm, idx_hbm, out_hbm, idx_v, rows_a, rows_b,
                 sga, sgb, swa, swb):
        wid = lax.axis_index("s") * SC_CORES + lax.axis_index("c")
        base = wid * per_worker
        pltpu.sync_copy(idx_hbm.at[pl.ds(base, per_worker)], idx_v)

        @pl.loop(0, niter // 2)
        def _(k):
            c0 = 2 * k * CHUNK
            c1 = c0 + CHUNK

            @pl.when(k > 0)
            def _():
                pltpu.make_async_copy(rows_a, out_hbm.at[pl.ds(base, CHUNK)],
                                      swa).wait()
                pltpu.make_async_copy(rows_b, out_hbm.at[pl.ds(base, CHUNK)],
                                      swb).wait()

            ha = pltpu.async_copy(table_hbm.at[idx_v.at[pl.ds(c0, CHUNK)]],
                                  rows_a, sga)
            hb = pltpu.async_copy(table_hbm.at[idx_v.at[pl.ds(c1, CHUNK)]],
                                  rows_b, sgb)
            ha.wait()
            pltpu.async_copy(rows_a, out_hbm.at[pl.ds(base + c0, CHUNK)], swa)
            hb.wait()
            pltpu.async_copy(rows_b, out_hbm.at[pl.ds(base + c1, CHUNK)], swb)

        pltpu.make_async_copy(rows_a, out_hbm.at[pl.ds(base, CHUNK)],
                              swa).wait()
        pltpu.make_async_copy(rows_b, out_hbm.at[pl.ds(base, CHUNK)],
                              swb).wait()

    return gather_k


def _make_scatter(nd_pad, ep):
    half = HID // 2
    e_sub = ep // SC_SUBCORES
    niter = e_sub // CHUNK
    rows_sub = nd_pad // SC_SUBCORES
    mesh = plsc.VectorSubcoreMesh(core_axis_name="c", subcore_axis_name="s")

    @functools.partial(
        pl.kernel,
        out_type=jax.ShapeDtypeStruct((nd_pad, HID), jnp.float32),
        mesh=mesh,
        scratch_types=[pltpu.VMEM((CHUNK,), jnp.int32),
                       pltpu.VMEM((CHUNK,), jnp.int32),
                       pltpu.VMEM((CHUNK, half), jnp.float32),
                       pltpu.VMEM((CHUNK, half), jnp.float32),
                       pltpu.VMEM_SHARED((nd_pad, half), jnp.float32),
                       pltpu.SemaphoreType.DMA,
                       pltpu.SemaphoreType.DMA,
                       pltpu.SemaphoreType.DMA,
                       pltpu.SemaphoreType.DMA,
                       pltpu.SemaphoreType.DMA,
                       pltpu.SemaphoreType.DMA],
    )
    def scatter_k(m_hbm, di_hbm, zeros_hbm, out_hbm, idx_a, idx_b, rows_a,
                  rows_b, acc_sh, sia, sib, sla, slb, ssa, ssb):
        cid = lax.axis_index("c")
        sid = lax.axis_index("s")
        r0 = sid * rows_sub
        col0 = cid * half
        pltpu.sync_copy(zeros_hbm.at[pl.ds(r0, rows_sub)],
                        acc_sh.at[pl.ds(r0, rows_sub)])
        plsc.subcore_barrier()

        @pl.loop(0, niter // 2)
        def _(k):
            off = sid * e_sub + 2 * k * CHUNK
            hia = pltpu.async_copy(di_hbm.at[pl.ds(off, CHUNK)], idx_a, sia)
            hla = pltpu.async_copy(
                m_hbm.at[pl.ds(off, CHUNK), pl.ds(col0, half)], rows_a, sla)
            hib = pltpu.async_copy(di_hbm.at[pl.ds(off + CHUNK, CHUNK)],
                                   idx_b, sib)
            hlb = pltpu.async_copy(
                m_hbm.at[pl.ds(off + CHUNK, CHUNK), pl.ds(col0, half)],
                rows_b, slb)
            hia.wait()
            hla.wait()
            hsa = pltpu.async_copy(rows_a, acc_sh.at[idx_a], ssa, add=True)
            hib.wait()
            hlb.wait()
            hsb = pltpu.async_copy(rows_b, acc_sh.at[idx_b], ssb, add=True)
            hsa.wait()
            hsb.wait()

        plsc.subcore_barrier()
        pltpu.sync_copy(acc_sh.at[pl.ds(r0, rows_sub)],
                        out_hbm.at[pl.ds(r0, rows_sub), pl.ds(col0, half)])

    return scatter_k


_EPAD = {k: _ceil_to(v, PAD_UNIT) for k, v in EDGE_N.items()}
_GATHER = {k: _make_gather(ep) for k, ep in _EPAD.items()}
_SCATTER = {name: _make_scatter(NPAD[dt], _EPAD[name])
            for name, _, dt in CONVS}


def kernel(x_base, x_centroid, edge_attr_bb, edge_attr_bc, edge_attr_cc,
           edge_attr_cb, edge_weight_bb, edge_weight_bc, edge_weight_cc,
           edge_weight_cb, src_bb, dst_bb, src_bc, dst_bc, src_cc, dst_cc,
           src_cb, dst_cb, W_src, b_src, W_dst, b_dst, W_edge, b_edge, eps,
           W_mlp, b_mlp, ln_g, ln_b):
    ea = {"bb": edge_attr_bb, "bc": edge_attr_bc, "cc": edge_attr_cc,
          "cb": edge_attr_cb}
    ew = {"bb": edge_weight_bb, "bc": edge_weight_bc, "cc": edge_weight_cc,
          "cb": edge_weight_cb}
    si = {"bb": src_bb, "bc": src_bc, "cc": src_cc, "cb": src_cb}
    di = {"bb": dst_bb, "bc": dst_bc, "cc": dst_cc, "cb": dst_cb}

    for name in EDGE_N:
        e = EDGE_N[name]
        ep = _EPAD[name]
        pad = ep - e
        ea[name] = jnp.pad(ea[name], ((0, pad), (0, 0)))
        ew[name] = jnp.pad(ew[name], (0, pad)).reshape(ep, 1)
        si[name] = jnp.pad(si[name], (0, pad))
        di[name] = jnp.pad(di[name], (0, pad))

    zeros = {"base": jnp.zeros((NPAD["base"], HID // 2), jnp.float32),
             "centroid": jnp.zeros((NPAD["centroid"], HID // 2), jnp.float32)}

    xmap = {"base": x_base, "centroid": x_centroid}
    for l in range(L):
        outs = {"base": [], "centroid": []}
        for c, (name, st, dt) in enumerate(CONVS):
            xs = xmap[st]
            xd = xmap[dt]
            nd = NNODE[dt]
            blk_s = 1000 if xs.shape[0] == NB else NCN
            blk_d = 1000 if nd == NB else NCN
            hsrc = _lin(xs, W_src[l, c], b_src[l, c], blk_s)
            hd = _lin(xd, W_dst[l, c] * (1.0 + eps[l, c]), b_dst[l, c], blk_d)
            g = _GATHER[name](hsrc, si[name])
            m = _msg(g, ea[name], ew[name], W_edge[l, c], b_edge[l, c], 2048)
            aggr = _SCATTER[name](m, di[name], zeros[dt])[:nd]
            out = _post(aggr, hd, W_mlp[l, c, 0], b_mlp[l, c, 0], ln_g[l, c],
                        ln_b[l, c], W_mlp[l, c, 1], b_mlp[l, c, 1], blk_d)
            outs[dt].append(out)
        xmap = {k: _combine(xmap[k], outs[k][0], outs[k][1],
                            1000 if NNODE[k] == NB else NCN)
                for k in xmap}
    return jnp.concatenate([xmap["base"], xmap["centroid"]], axis=0)
